# Initial kernel scaffold; baseline (speedup 1.0000x reference)
import jax
import jax.numpy as jnp
from jax.experimental import pallas as pl


def kernel(x, table):
    raise NotImplementedError("write your pallas kernel here")


# speedup vs baseline: 2.4026x; 2.4026x over previous
import functools
import math

import jax
import jax.numpy as jnp
from jax import lax
from jax.experimental import pallas as pl
from jax.experimental.pallas import tpu as pltpu
from jax.experimental.pallas import tpu_sc as plsc

VOCAB = 100000
EMBED_DIM = 64
LANES = 16

ROWS_PER_CHUNK = 2


def _position_encoding(max_len, d_model):
    position = jnp.arange(max_len, dtype=jnp.float32)[:, None]
    div_term = jnp.exp(
        jnp.arange(0, d_model, 2, dtype=jnp.float32) * (-math.log(10000.0) / d_model)
    )
    pe = jnp.zeros((max_len, d_model), dtype=jnp.float32)
    pe = pe.at[:, 0::2].set(jnp.sin(position * div_term))
    pe = pe.at[:, 1::2].set(jnp.cos(position * div_term))
    return pe


def kernel(x, table):
    B, S = x.shape
    V, D = table.shape
    assert D == EMBED_DIM

    info = plsc.get_sparse_core_info()
    NC, NS = info.num_cores, info.num_subcores
    NW = NC * NS

    total = B * S
    chunk = ROWS_PER_CHUNK * S
    per_worker = total // NW
    assert total % NW == 0 and per_worker % chunk == 0
    n_chunks = per_worker // chunk
    n_slices = (chunk * D) // LANES

    x_flat = x.reshape(total).astype(jnp.int32)
    pe = _position_encoding(S, D)
    pe_rep = jnp.tile(pe, (ROWS_PER_CHUNK, 1))

    mesh = plsc.VectorSubcoreMesh(core_axis_name="c", subcore_axis_name="s")

    @functools.partial(
        pl.kernel,
        mesh=mesh,
        compiler_params=pltpu.CompilerParams(use_tc_tiling_on_sc=False),
        out_type=jax.ShapeDtypeStruct((total, D), jnp.float32),
        scratch_types=[
            pltpu.VMEM((chunk,), jnp.int32),
            pltpu.VMEM((chunk, D), jnp.float32),
            pltpu.VMEM((chunk, D), jnp.float32),
            pltpu.SemaphoreType.DMA,
        ],
    )
    def _sc_kernel(x_hbm, pe_hbm, table_hbm, out_hbm, idx_v, pe_v, rows_v, sem):
        wid = lax.axis_index("s") * NC + lax.axis_index("c")
        worker_base = wid * per_worker

        pltpu.sync_copy(pe_hbm, pe_v)

        def chunk_body(c, _):
            base = worker_base + c * chunk
            pltpu.sync_copy(x_hbm.at[pl.ds(base, chunk)], idx_v)
            pltpu.async_copy(table_hbm.at[idx_v], rows_v, sem).wait()

            def add_body(i, _):
                r = i // (D // LANES)
                col = (i % (D // LANES)) * LANES
                sl = (r, pl.ds(col, LANES))
                rows_v[sl] = rows_v[sl] + pe_v[sl]
                return 0

            lax.fori_loop(0, n_slices, add_body, 0, unroll=4)
            pltpu.sync_copy(rows_v, out_hbm.at[pl.ds(base, chunk)])
            return 0

        lax.fori_loop(0, n_chunks, chunk_body, 0)

    out = _sc_kernel(x_flat, pe_rep, table)
    return out.reshape(B, S, D)


# speedup vs baseline: 3.3688x; 1.4021x over previous
import functools
import math

import jax
import jax.numpy as jnp
from jax import lax
from jax.experimental import pallas as pl
from jax.experimental.pallas import tpu as pltpu
from jax.experimental.pallas import tpu_sc as plsc

VOCAB = 100000
EMBED_DIM = 64
LANES = 16

ROWS_PER_CHUNK = 2


def _position_encoding(max_len, d_model):
    position = jnp.arange(max_len, dtype=jnp.float32)[:, None]
    div_term = jnp.exp(
        jnp.arange(0, d_model, 2, dtype=jnp.float32) * (-math.log(10000.0) / d_model)
    )
    pe = jnp.zeros((max_len, d_model), dtype=jnp.float32)
    pe = pe.at[:, 0::2].set(jnp.sin(position * div_term))
    pe = pe.at[:, 1::2].set(jnp.cos(position * div_term))
    return pe


def kernel(x, table):
    B, S = x.shape
    V, D = table.shape
    assert D == EMBED_DIM

    info = plsc.get_sparse_core_info()
    NC, NS = info.num_cores, info.num_subcores
    NW = NC * NS

    total = B * S
    chunk = ROWS_PER_CHUNK * S
    per_worker = total // NW
    assert total % NW == 0 and per_worker % chunk == 0
    n_chunks = per_worker // chunk
    n_slices = (chunk * D) // LANES

    x_flat = x.reshape(total).astype(jnp.int32)
    pe = _position_encoding(S, D)
    pe_rep = jnp.tile(pe, (ROWS_PER_CHUNK, 1))

    mesh = plsc.VectorSubcoreMesh(core_axis_name="c", subcore_axis_name="s")

    @functools.partial(
        pl.kernel,
        mesh=mesh,
        compiler_params=pltpu.CompilerParams(use_tc_tiling_on_sc=False),
        out_type=jax.ShapeDtypeStruct((total, D), jnp.float32),
        scratch_types=[
            pltpu.VMEM((chunk,), jnp.int32),
            pltpu.VMEM((chunk, D), jnp.float32),
            pltpu.VMEM((chunk, D), jnp.float32),
            pltpu.SemaphoreType.DMA,
        ],
    )
    def _sc_kernel(x_hbm, pe_hbm, table_hbm, out_hbm,
                   idx_v, pe_v, rows_v, sem):
        wid = lax.axis_index("s") * NC + lax.axis_index("c")
        worker_base = wid * per_worker

        pltpu.sync_copy(pe_hbm, pe_v)

        def chunk_body(c, _):
            base = worker_base + c * chunk
            pltpu.sync_copy(x_hbm.at[pl.ds(base, chunk)], idx_v)
            pltpu.async_copy(table_hbm.at[idx_v], rows_v, sem).wait()

            def row_body(r, _):
                for c4 in range(D // LANES):
                    sl = (r, pl.ds(c4 * LANES, LANES))
                    plsc.addupdate(rows_v.at[sl], pe_v[sl])
                return 0

            lax.fori_loop(0, chunk, row_body, 0, unroll=2)
            pltpu.sync_copy(rows_v, out_hbm.at[pl.ds(base, chunk)])
            return 0

        lax.fori_loop(0, n_chunks, chunk_body, 0)

    out = _sc_kernel(x_flat, pe_rep, table)
    return out.reshape(B, S, D)


# speedup vs baseline: 3.8395x; 1.1397x over previous
import functools
import math

import jax
import jax.numpy as jnp
from jax import lax
from jax.experimental import pallas as pl
from jax.experimental.pallas import tpu as pltpu
from jax.experimental.pallas import tpu_sc as plsc

VOCAB = 100000
EMBED_DIM = 64
LANES = 16

ROWS_PER_CHUNK = 1
NBUF = 4


def _position_encoding(max_len, d_model):
    position = jnp.arange(max_len, dtype=jnp.float32)[:, None]
    div_term = jnp.exp(
        jnp.arange(0, d_model, 2, dtype=jnp.float32) * (-math.log(10000.0) / d_model)
    )
    pe = jnp.zeros((max_len, d_model), dtype=jnp.float32)
    pe = pe.at[:, 0::2].set(jnp.sin(position * div_term))
    pe = pe.at[:, 1::2].set(jnp.cos(position * div_term))
    return pe


def kernel(x, table):
    B, S = x.shape
    V, D = table.shape
    assert D == EMBED_DIM

    info = plsc.get_sparse_core_info()
    NC, NS = info.num_cores, info.num_subcores
    NW = NC * NS

    total = B * S
    chunk = ROWS_PER_CHUNK * S
    per_worker = total // NW
    assert total % NW == 0 and per_worker % (chunk * NBUF) == 0
    n_groups = per_worker // (chunk * NBUF)

    x_flat = x.reshape(total).astype(jnp.int32)
    pe = _position_encoding(S, D)
    pe_rep = jnp.tile(pe, (ROWS_PER_CHUNK, 1))

    mesh = plsc.VectorSubcoreMesh(core_axis_name="c", subcore_axis_name="s")

    @functools.partial(
        pl.kernel,
        mesh=mesh,
        compiler_params=pltpu.CompilerParams(use_tc_tiling_on_sc=False),
        out_type=jax.ShapeDtypeStruct((total, D), jnp.float32),
        scratch_types=[
            pltpu.VMEM((per_worker,), jnp.int32),
            pltpu.VMEM((chunk, D), jnp.float32),
            pltpu.VMEM((NBUF, chunk, D), jnp.float32),
            pltpu.SemaphoreType.DMA((NBUF,)),
            pltpu.SemaphoreType.DMA((NBUF,)),
        ],
    )
    def _sc_kernel(x_hbm, pe_hbm, table_hbm, out_hbm,
                   idx_v, pe_v, rows_v, gsem, osem):
        wid = lax.axis_index("s") * NC + lax.axis_index("c")
        worker_base = wid * per_worker

        pltpu.sync_copy(pe_hbm, pe_v)
        pltpu.sync_copy(x_hbm.at[pl.ds(worker_base, per_worker)], idx_v)

        def group_body(g, _):
            gbase = g * (chunk * NBUF)
            for j in range(NBUF):
                pltpu.async_copy(
                    table_hbm.at[idx_v.at[pl.ds(gbase + j * chunk, chunk)]],
                    rows_v.at[j], gsem.at[j])
            for j in range(NBUF):
                pltpu.make_async_copy(
                    table_hbm.at[idx_v.at[pl.ds(gbase + j * chunk, chunk)]],
                    rows_v.at[j], gsem.at[j]).wait()

                def row_body(r, _):
                    for c4 in range(D // LANES):
                        sl = (r, pl.ds(c4 * LANES, LANES))
                        plsc.addupdate(rows_v.at[j].at[sl], pe_v[sl])
                    return 0

                lax.fori_loop(0, chunk, row_body, 0, unroll=4)
                pltpu.async_copy(
                    rows_v.at[j],
                    out_hbm.at[pl.ds(worker_base + gbase + j * chunk, chunk)],
                    osem.at[j])
            for j in range(NBUF):
                pltpu.make_async_copy(
                    rows_v.at[j],
                    out_hbm.at[pl.ds(worker_base + gbase + j * chunk, chunk)],
                    osem.at[j]).wait()
            return 0

        lax.fori_loop(0, n_groups, group_body, 0)

    out = _sc_kernel(x_flat, pe_rep, table)
    return out.reshape(B, S, D)
